# baseline (speedup 1.0000x reference)
import functools

import jax
import jax.numpy as jnp
from jax import lax
from jax.experimental import pallas as pl
from jax.experimental.pallas import tpu as pltpu
from jax.experimental.pallas import tpu_sc as plsc

L = 4
N = 10000
E = 320000
D = 128

NC = 2
NS = 16
NW = NC * NS

GW = 128
EBLK = 2000
NBLK = 1000
ZR = 80
NZCH = N // ZR
NCH = E // GW

@functools.cache
def _vector_mesh():
    return plsc.VectorSubcoreMesh(core_axis_name="c", subcore_axis_name="s")


def _bmm(a, w):
    return jnp.dot(a.astype(jnp.bfloat16), w.astype(jnp.bfloat16),
                   preferred_element_type=jnp.float32)


def _project_body(n_ref, w_ref, out_ref):
    out_ref[0] = _bmm(n_ref[...], w_ref[0])


def _project(n, wsd):
    return pl.pallas_call(
        _project_body,
        grid=(2, N // NBLK),
        in_specs=[
            pl.BlockSpec((NBLK, D), lambda t, i: (i, 0)),
            pl.BlockSpec((1, D, D), lambda t, i: (t, 0, 0)),
        ],
        out_specs=pl.BlockSpec((1, NBLK, D), lambda t, i: (t, i, 0)),
        out_shape=jax.ShapeDtypeStruct((2, N, D), jnp.float32),
    )(n, wsd)


def _edge_body(e_ref, g1_ref, g2_ref, w1_ref, b1_ref, w2_ref, b2_ref,
               lng_ref, lnb_ref, o_ref):
    e = e_ref[...]
    pre = (_bmm(e, w1_ref[...])
           + g1_ref[...] + g2_ref[...] + b1_ref[...])
    h1 = pre * jax.nn.sigmoid(pre)
    h = _bmm(h1, w2_ref[...]) + b2_ref[...]
    m = jnp.mean(h, axis=-1, keepdims=True)
    v = jnp.mean((h - m) * (h - m), axis=-1, keepdims=True)
    hn = (h - m) * lax.rsqrt(v + 1e-5) * lng_ref[...] + lnb_ref[...]
    o_ref[...] = e + hn


def _edge_mlp(e, g, w1, b1, w2, b2, lng, lnb):
    row = lambda i: (i, 0)
    full = lambda i: (0, 0)
    return pl.pallas_call(
        _edge_body,
        grid=(E // EBLK,),
        in_specs=[
            pl.BlockSpec((EBLK, D), row),
            pl.BlockSpec((EBLK, D), row),
            pl.BlockSpec((EBLK, D), lambda i: (E // EBLK + i, 0)),
            pl.BlockSpec((D, D), full),
            pl.BlockSpec((1, D), full),
            pl.BlockSpec((D, D), full),
            pl.BlockSpec((1, D), full),
            pl.BlockSpec((1, D), full),
            pl.BlockSpec((1, D), full),
        ],
        out_specs=pl.BlockSpec((EBLK, D), row),
        out_shape=jax.ShapeDtypeStruct((E, D), jnp.float32),
    )(e, g, g, w1, b1.reshape(1, D), w2, b2.reshape(1, D),
      lng.reshape(1, D), lnb.reshape(1, D))


def _node_body(n_ref, a0_ref, a1_ref, w1a_ref, w1b_ref, b1_ref, w2_ref,
               b2_ref, lng_ref, lnb_ref, o_ref):
    n = n_ref[...]
    agg = a0_ref[...] + a1_ref[...]
    pre = (_bmm(n, w1a_ref[...]) + _bmm(agg, w1b_ref[...]) + b1_ref[...])
    h1 = pre * jax.nn.sigmoid(pre)
    h = _bmm(h1, w2_ref[...]) + b2_ref[...]
    m = jnp.mean(h, axis=-1, keepdims=True)
    v = jnp.mean((h - m) * (h - m), axis=-1, keepdims=True)
    hn = (h - m) * lax.rsqrt(v + 1e-5) * lng_ref[...] + lnb_ref[...]
    o_ref[...] = n + hn


def _node_mlp(n, a0, a1, w1a, w1b, b1, w2, b2, lng, lnb):
    row = lambda i: (i, 0)
    full = lambda i: (0, 0)
    return pl.pallas_call(
        _node_body,
        grid=(N // NBLK,),
        in_specs=[
            pl.BlockSpec((NBLK, D), row),
            pl.BlockSpec((NBLK, D), row),
            pl.BlockSpec((NBLK, D), row),
            pl.BlockSpec((D, D), full),
            pl.BlockSpec((D, D), full),
            pl.BlockSpec((1, D), full),
            pl.BlockSpec((D, D), full),
            pl.BlockSpec((1, D), full),
            pl.BlockSpec((1, D), full),
            pl.BlockSpec((1, D), full),
        ],
        out_specs=pl.BlockSpec((NBLK, D), row),
        out_shape=jax.ShapeDtypeStruct((N, D), jnp.float32),
    )(n, a0, a1, w1a, w1b, b1.reshape(1, D), w2, b2.reshape(1, D),
      lng.reshape(1, D), lnb.reshape(1, D))


@functools.cache
def _gather_kernel():
    @functools.partial(
        pl.kernel,
        out_type=jax.ShapeDtypeStruct((2 * E, D), jnp.float32),
        mesh=_vector_mesh(),
    )
    def _gather(tab_hbm, j_hbm, g_hbm):
        def body(j_v, g_v):
            pltpu.sync_copy(tab_hbm.at[j_v.at[0]], g_v)

        pltpu.emit_pipeline(
            body,
            grid=(2 * E // GW,),
            in_specs=[pl.BlockSpec((1, GW), lambda i: (0, i))],
            out_specs=[pl.BlockSpec((GW, D), lambda i: (i, 0))],
            core_axis_name=("c", "s"),
            dimension_semantics=(pltpu.PARALLEL,),
            trace_scopes=False,
        )(j_hbm, g_hbm)

    return _gather


@functools.cache
def _segsum_kernel():
    @functools.partial(
        pl.kernel,
        out_type=jax.ShapeDtypeStruct((NC, N, D), jnp.float32),
        mesh=_vector_mesh(),
        scratch_types=[
            pltpu.VMEM_SHARED((N, D), jnp.float32),
            pltpu.VMEM((ZR, D), jnp.float32),
            pltpu.VMEM((GW, D), jnp.float32),
            pltpu.VMEM((GW, D), jnp.float32),
            pltpu.VMEM((1, GW), jnp.int32),
            pltpu.VMEM((1, GW), jnp.int32),
            pltpu.SemaphoreType.DMA,
            pltpu.SemaphoreType.DMA,
            pltpu.SemaphoreType.DMA,
            pltpu.SemaphoreType.DMA,
        ],
    )
    def _segsum(e_hbm, di_hbm, out_hbm, agg_sh, zbuf, eb0, eb1, ib0, ib1,
                sem0, sem1, ssem0, ssem1):
        c = lax.axis_index("c")
        s = lax.axis_index("s")
        wid = s * NC + c
        ebufs = (eb0, eb1)
        ibufs = (ib0, ib1)
        sems = (sem0, sem1)
        ssems = (ssem0, ssem1)

        @pl.loop(0, ZR)
        def _(r):
            @pl.loop(0, D, step=16)
            def _(col):
                zbuf.at[pl.ds(r, 1), pl.ds(col, 16)][...] = jnp.zeros(
                    (1, 16), jnp.float32)

        @pl.loop(s, NZCH, step=NS)
        def _(k):
            pltpu.sync_copy(zbuf, agg_sh.at[pl.ds(k * ZR, ZR)])

        plsc.subcore_barrier()

        def _start(b, k):
            pltpu.async_copy(e_hbm.at[pl.ds(k * GW, GW)], ebufs[b], sems[b])
            pltpu.async_copy(di_hbm.at[k], ibufs[b], sems[b])

        def _wait(b, k):
            pltpu.make_async_copy(
                e_hbm.at[pl.ds(k * GW, GW)], ebufs[b], sems[b]).wait()
            pltpu.make_async_copy(di_hbm.at[k], ibufs[b], sems[b]).wait()

        def _scat_start(b):
            pltpu.async_copy(ebufs[b], agg_sh.at[ibufs[b].at[0]], ssems[b],
                             add=True)

        def _scat_wait(b):
            pltpu.make_async_copy(
                ebufs[b], agg_sh.at[ibufs[b].at[0]], ssems[b]).wait()

        _start(0, wid)

        @pl.loop(0, 40)
        def _(j):
            for b in range(2):
                j2 = 2 * j + b
                k = wid + NW * j2
                b1 = 1 - b

                @pl.when(k < NCH)
                def _():
                    _wait(b, k)
                    _scat_start(b)

                    @pl.when(j2 >= 1)
                    def _():
                        _scat_wait(b1)

                    @pl.when(k + NW < NCH)
                    def _():
                        _start(b1, k + NW)

        nmine = (NCH - 1 - wid) // NW + 1
        lastb = (nmine - 1) % 2

        @pl.when(lastb == 0)
        def _():
            _scat_wait(0)

        @pl.when(lastb == 1)
        def _():
            _scat_wait(1)

        plsc.subcore_barrier()

        @pl.loop(s, NZCH, step=NS)
        def _(k):
            r0 = k * ZR
            pltpu.sync_copy(agg_sh.at[pl.ds(r0, ZR)], zbuf)
            pltpu.sync_copy(zbuf, out_hbm.at[c, pl.ds(r0, ZR)])

    return _segsum


def kernel(edge_feats, node_feats, edge_index,
           edge_W1, edge_b1, edge_W2, edge_b2, edge_ln_g, edge_ln_b,
           node_W1, node_b1, node_W2, node_b2, node_ln_g, node_ln_b):
    src = edge_index[0]
    dst = edge_index[1]
    jidx = jnp.concatenate([src, dst + N]).reshape(1, 2 * E)
    dst3 = dst.reshape(NCH, 1, GW)
    e = edge_feats
    n = node_feats
    for i in range(L):
        w1 = edge_W1[i]
        psd = _project(n, w1[D:].reshape(2, D, D))
        g = _gather_kernel()(psd.reshape(2 * N, D), jidx)
        e = _edge_mlp(e, g, w1[:D], edge_b1[i], edge_W2[i], edge_b2[i],
                      edge_ln_g[i], edge_ln_b[i])
        aggp = _segsum_kernel()(e, dst3)
        n = _node_mlp(n, aggp[0], aggp[1], node_W1[i, :D], node_W1[i, D:],
                      node_b1[i], node_W2[i], node_b2[i],
                      node_ln_g[i], node_ln_b[i])
    return e, n


# speedup vs baseline: 1.1455x; 1.1455x over previous
import functools

import jax
import jax.numpy as jnp
from jax import lax
from jax.experimental import pallas as pl
from jax.experimental.pallas import tpu as pltpu
from jax.experimental.pallas import tpu_sc as plsc

L = 4
N = 10000
E = 320000
D = 128

NC = 2
NS = 16
NW = NC * NS

GW = 128
EBLK = 2000
NBLK = 1000
ZR = 80
NZCH = N // ZR
EH = E // 2
NCHH = EH // GW
EHBLK = EH // EBLK

@functools.cache
def _vector_mesh():
    return plsc.VectorSubcoreMesh(core_axis_name="c", subcore_axis_name="s")


def _bmm(a, w):
    return jnp.dot(a, w, preferred_element_type=jnp.float32)


def _project_body(n_ref, w_ref, out_ref):
    out_ref[0] = _bmm(n_ref[...], w_ref[0])


def _project(n, wsd):
    return pl.pallas_call(
        _project_body,
        grid=(2, N // NBLK),
        in_specs=[
            pl.BlockSpec((NBLK, D), lambda t, i: (i, 0)),
            pl.BlockSpec((1, D, D), lambda t, i: (t, 0, 0)),
        ],
        out_specs=pl.BlockSpec((1, NBLK, D), lambda t, i: (t, i, 0)),
        out_shape=jax.ShapeDtypeStruct((2, N, D), jnp.float32),
    )(n, wsd)


def _edge_body(e_ref, g1_ref, g2_ref, w1_ref, b1_ref, w2_ref, b2_ref,
               lng_ref, lnb_ref, o_ref):
    e = e_ref[...]
    pre = (_bmm(e, w1_ref[...])
           + g1_ref[...] + g2_ref[...] + b1_ref[...])
    h1 = pre * jax.nn.sigmoid(pre)
    h = _bmm(h1, w2_ref[...]) + b2_ref[...]
    m = jnp.mean(h, axis=-1, keepdims=True)
    v = jnp.mean((h - m) * (h - m), axis=-1, keepdims=True)
    hn = (h - m) * lax.rsqrt(v + 1e-5) * lng_ref[...] + lnb_ref[...]
    o_ref[...] = e + hn


def _edge_mlp(e, e_off, g, w1, b1, w2, b2, lng, lnb):
    row = lambda i: (i, 0)
    full = lambda i: (0, 0)
    return pl.pallas_call(
        _edge_body,
        grid=(EHBLK,),
        in_specs=[
            pl.BlockSpec((EBLK, D), lambda i: (e_off + i, 0)),
            pl.BlockSpec((EBLK, D), row),
            pl.BlockSpec((EBLK, D), lambda i: (EHBLK + i, 0)),
            pl.BlockSpec((D, D), full),
            pl.BlockSpec((1, D), full),
            pl.BlockSpec((D, D), full),
            pl.BlockSpec((1, D), full),
            pl.BlockSpec((1, D), full),
            pl.BlockSpec((1, D), full),
        ],
        out_specs=pl.BlockSpec((EBLK, D), row),
        out_shape=jax.ShapeDtypeStruct((EH, D), jnp.float32),
    )(e, g, g, w1, b1.reshape(1, D), w2, b2.reshape(1, D),
      lng.reshape(1, D), lnb.reshape(1, D))


def _node_body(n_ref, a0_ref, a1_ref, a2_ref, a3_ref, w1a_ref, w1b_ref,
               b1_ref, w2_ref, b2_ref, lng_ref, lnb_ref, o_ref):
    n = n_ref[...]
    agg = (a0_ref[0] + a1_ref[0]) + (a2_ref[0] + a3_ref[0])
    pre = (_bmm(n, w1a_ref[...]) + _bmm(agg, w1b_ref[...]) + b1_ref[...])
    h1 = pre * jax.nn.sigmoid(pre)
    h = _bmm(h1, w2_ref[...]) + b2_ref[...]
    m = jnp.mean(h, axis=-1, keepdims=True)
    v = jnp.mean((h - m) * (h - m), axis=-1, keepdims=True)
    hn = (h - m) * lax.rsqrt(v + 1e-5) * lng_ref[...] + lnb_ref[...]
    o_ref[...] = n + hn


def _node_mlp(n, aggpA, aggpB, w1a, w1b, b1, w2, b2, lng, lnb):
    row = lambda i: (i, 0)
    full = lambda i: (0, 0)
    p0 = lambda i: (0, i, 0)
    p1 = lambda i: (1, i, 0)
    return pl.pallas_call(
        _node_body,
        grid=(N // NBLK,),
        in_specs=[
            pl.BlockSpec((NBLK, D), row),
            pl.BlockSpec((1, NBLK, D), p0),
            pl.BlockSpec((1, NBLK, D), p1),
            pl.BlockSpec((1, NBLK, D), p0),
            pl.BlockSpec((1, NBLK, D), p1),
            pl.BlockSpec((D, D), full),
            pl.BlockSpec((D, D), full),
            pl.BlockSpec((1, D), full),
            pl.BlockSpec((D, D), full),
            pl.BlockSpec((1, D), full),
            pl.BlockSpec((1, D), full),
            pl.BlockSpec((1, D), full),
        ],
        out_specs=pl.BlockSpec((NBLK, D), row),
        out_shape=jax.ShapeDtypeStruct((N, D), jnp.float32),
    )(n, aggpA, aggpA, aggpB, aggpB, w1a, w1b, b1.reshape(1, D), w2,
      b2.reshape(1, D), lng.reshape(1, D), lnb.reshape(1, D))


@functools.cache
def _gather_kernel():
    @functools.partial(
        pl.kernel,
        out_type=jax.ShapeDtypeStruct((E, D), jnp.float32),
        mesh=_vector_mesh(),
    )
    def _gather(tab_hbm, j_hbm, g_hbm):
        def body(j_v, g_v):
            pltpu.sync_copy(tab_hbm.at[j_v.at[0]], g_v)

        pltpu.emit_pipeline(
            body,
            grid=(E // GW,),
            in_specs=[pl.BlockSpec((1, GW), lambda i: (0, i))],
            out_specs=[pl.BlockSpec((GW, D), lambda i: (i, 0))],
            core_axis_name=("c", "s"),
            dimension_semantics=(pltpu.PARALLEL,),
            trace_scopes=False,
        )(j_hbm, g_hbm)

    return _gather


@functools.cache
def _segsum_kernel():
    @functools.partial(
        pl.kernel,
        out_type=jax.ShapeDtypeStruct((NC, N, D), jnp.float32),
        mesh=_vector_mesh(),
        scratch_types=[
            pltpu.VMEM_SHARED((N, D), jnp.float32),
            pltpu.VMEM((ZR, D), jnp.float32),
            pltpu.VMEM((GW, D), jnp.float32),
            pltpu.VMEM((GW, D), jnp.float32),
            pltpu.VMEM((1, GW), jnp.int32),
            pltpu.VMEM((1, GW), jnp.int32),
            pltpu.SemaphoreType.DMA,
            pltpu.SemaphoreType.DMA,
            pltpu.SemaphoreType.DMA,
            pltpu.SemaphoreType.DMA,
        ],
    )
    def _segsum(e_hbm, di_hbm, out_hbm, agg_sh, zbuf, eb0, eb1, ib0, ib1,
                sem0, sem1, ssem0, ssem1):
        c = lax.axis_index("c")
        s = lax.axis_index("s")
        wid = s * NC + c
        ebufs = (eb0, eb1)
        ibufs = (ib0, ib1)
        sems = (sem0, sem1)
        ssems = (ssem0, ssem1)

        @pl.loop(0, ZR)
        def _(r):
            @pl.loop(0, D, step=16)
            def _(col):
                zbuf.at[pl.ds(r, 1), pl.ds(col, 16)][...] = jnp.zeros(
                    (1, 16), jnp.float32)

        @pl.loop(s, NZCH, step=NS)
        def _(k):
            pltpu.sync_copy(zbuf, agg_sh.at[pl.ds(k * ZR, ZR)])

        plsc.subcore_barrier()

        def _start(b, k):
            pltpu.async_copy(e_hbm.at[pl.ds(k * GW, GW)], ebufs[b], sems[b])
            pltpu.async_copy(di_hbm.at[k], ibufs[b], sems[b])

        def _wait(b, k):
            pltpu.make_async_copy(
                e_hbm.at[pl.ds(k * GW, GW)], ebufs[b], sems[b]).wait()
            pltpu.make_async_copy(di_hbm.at[k], ibufs[b], sems[b]).wait()

        def _scat_start(b):
            pltpu.async_copy(ebufs[b], agg_sh.at[ibufs[b].at[0]], ssems[b],
                             add=True)

        def _scat_wait(b):
            pltpu.make_async_copy(
                ebufs[b], agg_sh.at[ibufs[b].at[0]], ssems[b]).wait()

        _start(0, wid)

        @pl.loop(0, 20)
        def _(j):
            for b in range(2):
                j2 = 2 * j + b
                k = wid + NW * j2
                b1 = 1 - b

                @pl.when(k < NCHH)
                def _():
                    _wait(b, k)
                    _scat_start(b)

                    @pl.when(j2 >= 1)
                    def _():
                        _scat_wait(b1)

                    @pl.when(k + NW < NCHH)
                    def _():
                        _start(b1, k + NW)

        nmine = (NCHH - 1 - wid) // NW + 1
        lastb = (nmine - 1) % 2

        @pl.when(lastb == 0)
        def _():
            _scat_wait(0)

        @pl.when(lastb == 1)
        def _():
            _scat_wait(1)

        plsc.subcore_barrier()

        @pl.loop(s, NZCH, step=NS)
        def _(k):
            r0 = k * ZR
            pltpu.sync_copy(agg_sh.at[pl.ds(r0, ZR)], zbuf)
            pltpu.sync_copy(zbuf, out_hbm.at[c, pl.ds(r0, ZR)])

    return _segsum


def kernel(edge_feats, node_feats, edge_index,
           edge_W1, edge_b1, edge_W2, edge_b2, edge_ln_g, edge_ln_b,
           node_W1, node_b1, node_W2, node_b2, node_ln_g, node_ln_b):
    src = edge_index[0]
    dst = edge_index[1]
    srcA, srcB = src[:EH], src[EH:]
    dstA, dstB = dst[:EH], dst[EH:]
    jidxA = jnp.concatenate([srcA, dstA + N]).reshape(1, E)
    jidxB = jnp.concatenate([srcB, dstB + N]).reshape(1, E)
    dstA3 = dstA.reshape(NCHH, 1, GW)
    dstB3 = dstB.reshape(NCHH, 1, GW)
    n = node_feats
    eA = eB = None
    for i in range(L):
        w1 = edge_W1[i]
        psd = _project(n, w1[D:].reshape(2, D, D))
        tab = psd.reshape(2 * N, D)
        gA = _gather_kernel()(tab, jidxA)
        gB = _gather_kernel()(tab, jidxB)
        ew = (w1[:D], edge_b1[i], edge_W2[i], edge_b2[i],
              edge_ln_g[i], edge_ln_b[i])
        if i == 0:
            eA = _edge_mlp(edge_feats, 0, gA, *ew)
            eB = _edge_mlp(edge_feats, EHBLK, gB, *ew)
        else:
            eA = _edge_mlp(eA, 0, gA, *ew)
            eB = _edge_mlp(eB, 0, gB, *ew)
        aggpA = _segsum_kernel()(eA, dstA3)
        aggpB = _segsum_kernel()(eB, dstB3)
        n = _node_mlp(n, aggpA, aggpB, node_W1[i, :D], node_W1[i, D:],
                      node_b1[i], node_W2[i], node_b2[i],
                      node_ln_g[i], node_ln_b[i])
    return jnp.concatenate([eA, eB], axis=0), n
